# baseline (speedup 1.0000x reference)
import functools

import jax
import jax.numpy as jnp
from jax import lax
from jax.experimental import pallas as pl
from jax.experimental.pallas import tpu as pltpu
from jax.experimental.pallas import tpu_sc as plsc

NC = 2
NS = 16
NW = NC * NS
LANES = 16


def _tc_prompt_proj(x_ref, w_ref, prompt_ref, bias_ref, outx_ref, pt_ref):
    xb = x_ref[...]
    outx_ref[...] = xb + prompt_ref[...]
    pt = lax.dot_general(
        w_ref[...], xb, (((1,), (1,)), ((), ())),
        preferred_element_type=jnp.float32,
    )
    pt_ref[...] = pt + bias_ref[...][:, 0:1]


def _tc_anchor_matmul(bt_ref, anc_ref, out_ref):
    out_ref[...] = lax.dot_general(
        bt_ref[...], anc_ref[...], (((0,), (0,)), ((), ())),
        preferred_element_type=jnp.float32,
    )


def _tc_anchor_matmul_alias(bt_ref, anc_ref, ep_ref, out_ref):
    del ep_ref
    out_ref[...] = lax.dot_general(
        bt_ref[...], anc_ref[...], (((0,), (0,)), ((), ())),
        preferred_element_type=jnp.float32,
    )


def _sc_edge_softmax(A, N, E, H, C, eph, eoff, pt_hbm, ei_hbm, out_hbm,
                     table, sidx, didx, obuf):
    cid = lax.axis_index("c")
    sid = lax.axis_index("s")
    wid = sid * NC + cid
    pltpu.sync_copy(pt_hbm, table)
    base0 = wid * eph

    def chunk_body(k, carry):
        base = base0 + k * C
        abs_base = eoff + base
        pltpu.sync_copy(ei_hbm.at[pl.ds(abs_base, C)], sidx)
        pltpu.sync_copy(ei_hbm.at[pl.ds(E + abs_base, C)], didx)

        def do_group(off):
            si = sidx[pl.ds(off, LANES)]
            di = didx[pl.ds(off, LANES)]
            logits = []
            for a in range(A):
                ls = plsc.load_gather(table, [si + jnp.int32(a * N)])
                ld = plsc.load_gather(table, [di + jnp.int32((A + a) * N)])
                l = ls + ld
                logits.append(jnp.maximum(l, 0.01 * l))
            m = logits[0]
            for a in range(1, A):
                m = jnp.maximum(m, logits[a])
            exps = [jnp.exp(l - m) for l in logits]
            tot = exps[0]
            for a in range(1, A):
                tot = tot + exps[a]
            r = 1.0 / tot
            for a in range(A):
                obuf[pl.ds(a * C + off, LANES)] = exps[a] * r

        def group_body(g, carry2):
            do_group(g * LANES)
            return carry2

        lax.fori_loop(0, C // LANES, group_body, 0)
        if C % LANES:
            do_group(C - LANES)
        for a in range(A):
            pltpu.sync_copy(obuf.at[pl.ds(a * C, C)],
                            out_hbm.at[pl.ds(a * H + base, C)])
        return carry

    lax.fori_loop(0, eph // C, chunk_body, 0)


def kernel(x, edge_index, node_prompt, anchor_prompt, w_weight, w_bias, layer):
    N, D = x.shape
    E = edge_index.shape[1]
    A = w_weight.shape[0]

    w_cat = jnp.concatenate([w_weight[:, :D], w_weight[:, D:]], axis=0)
    bias_cat = jnp.concatenate([w_bias, jnp.zeros((A,), jnp.float32)])
    bias_cat = jnp.broadcast_to(bias_cat[:, None], (2 * A, 128))

    outx, pt = pl.pallas_call(
        _tc_prompt_proj,
        out_shape=(
            jax.ShapeDtypeStruct((N, D), jnp.float32),
            jax.ShapeDtypeStruct((2 * A, N), jnp.float32),
        ),
    )(x, w_cat, node_prompt, bias_cat)

    H = E // 2
    eph = H // NW
    C = 1000
    mesh = plsc.VectorSubcoreMesh(core_axis_name="c", subcore_axis_name="s")
    pt_flat = pt.reshape(2 * A * N)
    ei_flat = edge_index.reshape(2 * E)

    def make_sc(eoff):
        return pl.kernel(
            functools.partial(_sc_edge_softmax, A, N, E, H, C, eph, eoff),
            out_type=jax.ShapeDtypeStruct((A * H,), jnp.float32),
            mesh=mesh,
            compiler_params=pltpu.CompilerParams(needs_layout_passes=False),
            scratch_types=[
                pltpu.VMEM((2 * A * N,), jnp.float32),
                pltpu.VMEM((C,), jnp.int32),
                pltpu.VMEM((C,), jnp.int32),
                pltpu.VMEM((A * C,), jnp.float32),
            ],
        )

    bt0 = make_sc(0)(pt_flat, ei_flat).reshape(A, H)
    bt1 = make_sc(H)(pt_flat, ei_flat).reshape(A, H)

    EB = 6400
    nb = H // EB
    ep0 = pl.pallas_call(
        _tc_anchor_matmul,
        grid=(nb,),
        in_specs=[
            pl.BlockSpec((A, EB), lambda i: (0, i)),
            pl.BlockSpec((A, D), lambda i: (0, 0)),
        ],
        out_specs=pl.BlockSpec((EB, D), lambda i: (i, 0)),
        out_shape=jax.ShapeDtypeStruct((E, D), jnp.float32),
    )(bt0, anchor_prompt)

    edge_prompt = pl.pallas_call(
        _tc_anchor_matmul_alias,
        grid=(nb,),
        in_specs=[
            pl.BlockSpec((A, EB), lambda i: (0, i)),
            pl.BlockSpec((A, D), lambda i: (0, 0)),
            pl.BlockSpec(memory_space=pltpu.MemorySpace.HBM),
        ],
        out_specs=pl.BlockSpec((EB, D), lambda i: (i + nb, 0)),
        out_shape=jax.ShapeDtypeStruct((E, D), jnp.float32),
        input_output_aliases={2: 0},
    )(bt1, anchor_prompt, ep0)

    return (outx, edge_prompt)


# speedup vs baseline: 1.0942x; 1.0942x over previous
import functools

import jax
import jax.numpy as jnp
from jax import lax
from jax.experimental import pallas as pl
from jax.experimental.pallas import tpu as pltpu
from jax.experimental.pallas import tpu_sc as plsc

NC = 2
NS = 16
NW = NC * NS
LANES = 16


def _tc_proj(x_ref, w_ref, bias_ref, pt_ref):
    pt = lax.dot_general(
        w_ref[...], x_ref[...], (((1,), (1,)), ((), ())),
        preferred_element_type=jnp.float32,
    )
    pt_ref[...] = pt + bias_ref[...][:, 0:1]


def _tc_anchor_matmul(bt_ref, anc_ref, x_ref, prompt_ref, out_ref, outx_ref):
    out_ref[...] = lax.dot_general(
        bt_ref[...], anc_ref[...], (((0,), (0,)), ((), ())),
        preferred_element_type=jnp.float32,
    )
    outx_ref[...] = x_ref[...] + prompt_ref[...]


def _sc_edge_softmax(A, N, E, H, C, eph, eoff, pt_hbm, ei_hbm, out_hbm,
                     table, sidx, didx, obuf):
    cid = lax.axis_index("c")
    sid = lax.axis_index("s")
    wid = sid * NC + cid
    pltpu.sync_copy(pt_hbm, table)
    base0 = wid * eph

    def chunk_body(k, carry):
        base = base0 + k * C
        abs_base = eoff + base
        pltpu.sync_copy(ei_hbm.at[pl.ds(abs_base, C)], sidx)
        pltpu.sync_copy(ei_hbm.at[pl.ds(E + abs_base, C)], didx)

        def do_group(off):
            si = sidx[pl.ds(off, LANES)]
            di = didx[pl.ds(off, LANES)]
            logits = []
            for a in range(A):
                ls = plsc.load_gather(table, [si + jnp.int32(a * N)])
                ld = plsc.load_gather(table, [di + jnp.int32((A + a) * N)])
                l = ls + ld
                logits.append(jnp.maximum(l, 0.01 * l))
            m = logits[0]
            for a in range(1, A):
                m = jnp.maximum(m, logits[a])
            exps = [jnp.exp(l - m) for l in logits]
            tot = exps[0]
            for a in range(1, A):
                tot = tot + exps[a]
            r = 1.0 / tot
            for a in range(A):
                obuf[pl.ds(a * C + off, LANES)] = exps[a] * r

        def group_body(g, carry2):
            do_group(g * LANES)
            return carry2

        lax.fori_loop(0, C // LANES, group_body, 0)
        if C % LANES:
            do_group(C - LANES)
        for a in range(A):
            pltpu.sync_copy(obuf.at[pl.ds(a * C, C)],
                            out_hbm.at[pl.ds(a * H + base, C)])
        return carry

    lax.fori_loop(0, eph // C, chunk_body, 0)


def kernel(x, edge_index, node_prompt, anchor_prompt, w_weight, w_bias, layer):
    N, D = x.shape
    E = edge_index.shape[1]
    A = w_weight.shape[0]

    w_cat = jnp.concatenate([w_weight[:, :D], w_weight[:, D:]], axis=0)
    bias_cat = jnp.concatenate([w_bias, jnp.zeros((A,), jnp.float32)])
    bias_cat = jnp.broadcast_to(bias_cat[:, None], (2 * A, 128))

    pt = pl.pallas_call(
        _tc_proj,
        out_shape=jax.ShapeDtypeStruct((2 * A, N), jnp.float32),
    )(x, w_cat, bias_cat)

    epw = E // NW
    C = 2000
    mesh = plsc.VectorSubcoreMesh(core_axis_name="c", subcore_axis_name="s")
    sc_fn = pl.kernel(
        functools.partial(_sc_edge_softmax, A, N, E, E, C, epw, 0),
        out_type=jax.ShapeDtypeStruct((A * E,), jnp.float32),
        mesh=mesh,
        compiler_params=pltpu.CompilerParams(needs_layout_passes=False),
        scratch_types=[
            pltpu.VMEM((2 * A * N,), jnp.float32),
            pltpu.VMEM((C,), jnp.int32),
            pltpu.VMEM((C,), jnp.int32),
            pltpu.VMEM((A * C,), jnp.float32),
        ],
    )
    bt = sc_fn(pt.reshape(2 * A * N), edge_index.reshape(2 * E)).reshape(A, E)

    EB = 12800
    nb = E // EB
    XB = N // nb
    edge_prompt, outx = pl.pallas_call(
        _tc_anchor_matmul,
        grid=(nb,),
        in_specs=[
            pl.BlockSpec((A, EB), lambda i: (0, i)),
            pl.BlockSpec((A, D), lambda i: (0, 0)),
            pl.BlockSpec((XB, D), lambda i: (i, 0)),
            pl.BlockSpec((1, D), lambda i: (0, 0)),
        ],
        out_specs=(
            pl.BlockSpec((EB, D), lambda i: (i, 0)),
            pl.BlockSpec((XB, D), lambda i: (i, 0)),
        ),
        out_shape=(
            jax.ShapeDtypeStruct((E, D), jnp.float32),
            jax.ShapeDtypeStruct((N, D), jnp.float32),
        ),
    )(bt, anchor_prompt, x, node_prompt)

    return (outx, edge_prompt)


# speedup vs baseline: 1.1648x; 1.0646x over previous
import functools

import jax
import jax.numpy as jnp
from jax import lax
from jax.experimental import pallas as pl
from jax.experimental.pallas import tpu as pltpu
from jax.experimental.pallas import tpu_sc as plsc

NC = 2
NS = 16
NW = NC * NS
LANES = 16


def _tc_proj(x_ref, w_ref, bias_ref, pt_ref):
    pt = lax.dot_general(
        w_ref[...], x_ref[...], (((1,), (1,)), ((), ())),
        preferred_element_type=jnp.float32,
    )
    pt_ref[...] = pt + bias_ref[...][:, 0:1]


def _tc_anchor_matmul(bt_ref, anc_ref, x_ref, prompt_ref, out_ref, outx_ref):
    out_ref[...] = lax.dot_general(
        bt_ref[...], anc_ref[...], (((0,), (0,)), ((), ())),
        preferred_element_type=jnp.float32,
    )
    outx_ref[...] = x_ref[...] + prompt_ref[...]


def _sc_edge_softmax(A, N, E, C, epw, pt_hbm, ei_hbm, out_hbm,
                     table, sidx, didx, obuf, sem_in, sem_out):
    cid = lax.axis_index("c")
    sid = lax.axis_index("s")
    wid = sid * NC + cid
    base0 = wid * epw
    nchunk = epw // C

    def start_in(k, b):
        base = base0 + k * C
        d0 = pltpu.make_async_copy(ei_hbm.at[pl.ds(base, C)],
                                   sidx[b], sem_in[b])
        d1 = pltpu.make_async_copy(ei_hbm.at[pl.ds(E + base, C)],
                                   didx[b], sem_in[b])
        d0.start()
        d1.start()
        return (d0, d1)

    def start_out(k, b):
        base = base0 + k * C
        ds = []
        for a in range(A):
            d = pltpu.make_async_copy(obuf[b].at[pl.ds(a * C, C)],
                                      out_hbm.at[pl.ds(a * E + base, C)],
                                      sem_out[b])
            d.start()
            ds.append(d)
        return ds

    in_d = {0: start_in(0, 0)}
    pltpu.sync_copy(pt_hbm, table)
    out_d = {}

    for k in range(nchunk):
        b = k % 2
        if k + 1 < nchunk:
            in_d[k + 1] = start_in(k + 1, 1 - b)
        for d in in_d.pop(k):
            d.wait()
        if k >= 2:
            for d in out_d.pop(k - 2):
                d.wait()

        def do_group(off):
            si = sidx[b][pl.ds(off, LANES)]
            di = didx[b][pl.ds(off, LANES)]
            logits = []
            for a in range(A):
                ls = plsc.load_gather(table, [si + jnp.int32(a * N)])
                ld = plsc.load_gather(table, [di + jnp.int32((A + a) * N)])
                l = ls + ld
                logits.append(jnp.maximum(l, 0.01 * l))
            m = logits[0]
            for a in range(1, A):
                m = jnp.maximum(m, logits[a])
            exps = [jnp.exp(l - m) for l in logits]
            tot = exps[0]
            for a in range(1, A):
                tot = tot + exps[a]
            r = 1.0 / tot
            for a in range(A):
                obuf[b][pl.ds(a * C + off, LANES)] = exps[a] * r

        def group_body(g, carry2):
            do_group(g * LANES)
            return carry2

        lax.fori_loop(0, C // LANES, group_body, 0)
        out_d[k] = start_out(k, b)

    for k in sorted(out_d):
        for d in out_d.pop(k):
            d.wait()


def kernel(x, edge_index, node_prompt, anchor_prompt, w_weight, w_bias, layer):
    N, D = x.shape
    E = edge_index.shape[1]
    A = w_weight.shape[0]

    w_cat = jnp.concatenate([w_weight[:, :D], w_weight[:, D:]], axis=0)
    bias_cat = jnp.concatenate([w_bias, jnp.zeros((A,), jnp.float32)])
    bias_cat = jnp.broadcast_to(bias_cat[:, None], (2 * A, 128))

    pt = pl.pallas_call(
        _tc_proj,
        out_shape=jax.ShapeDtypeStruct((2 * A, N), jnp.float32),
    )(x, w_cat, bias_cat)

    epw = E // NW
    C = 2000
    mesh = plsc.VectorSubcoreMesh(core_axis_name="c", subcore_axis_name="s")
    sc_fn = pl.kernel(
        functools.partial(_sc_edge_softmax, A, N, E, C, epw),
        out_type=jax.ShapeDtypeStruct((A * E,), jnp.float32),
        mesh=mesh,
        compiler_params=pltpu.CompilerParams(needs_layout_passes=False),
        scratch_types=[
            pltpu.VMEM((2 * A * N,), jnp.float32),
            [pltpu.VMEM((C,), jnp.int32)] * 2,
            [pltpu.VMEM((C,), jnp.int32)] * 2,
            [pltpu.VMEM((A * C,), jnp.float32)] * 2,
            [pltpu.SemaphoreType.DMA] * 2,
            [pltpu.SemaphoreType.DMA] * 2,
        ],
    )
    bt = sc_fn(pt.reshape(2 * A * N), edge_index.reshape(2 * E)).reshape(A, E)

    EB = 12800
    nb = E // EB
    XB = N // nb
    edge_prompt, outx = pl.pallas_call(
        _tc_anchor_matmul,
        grid=(nb,),
        in_specs=[
            pl.BlockSpec((A, EB), lambda i: (0, i)),
            pl.BlockSpec((A, D), lambda i: (0, 0)),
            pl.BlockSpec((XB, D), lambda i: (i, 0)),
            pl.BlockSpec((1, D), lambda i: (0, 0)),
        ],
        out_specs=(
            pl.BlockSpec((EB, D), lambda i: (i, 0)),
            pl.BlockSpec((XB, D), lambda i: (i, 0)),
        ),
        out_shape=(
            jax.ShapeDtypeStruct((E, D), jnp.float32),
            jax.ShapeDtypeStruct((N, D), jnp.float32),
        ),
    )(bt, anchor_prompt, x, node_prompt)

    return (outx, edge_prompt)
